# baseline (speedup 1.0000x reference)
import jax
import jax.numpy as jnp
from jax.experimental import pallas as pl


def _copy_body(pe_ref, out_ref):
    out_ref[...] = pe_ref[...]


def kernel(x, pe):
    seq_len = x.shape[1]
    d = pe.shape[2]
    pe2 = pe.reshape(pe.shape[1], d)
    block = 2048
    out = pl.pallas_call(
        _copy_body,
        grid=(seq_len // block,),
        in_specs=[pl.BlockSpec((block, d), lambda i: (i, 0))],
        out_specs=pl.BlockSpec((block, d), lambda i: (i, 0)),
        out_shape=jax.ShapeDtypeStruct((seq_len, d), pe.dtype),
    )(pe2)
    return out.reshape(1, seq_len, d)
